# Initial kernel scaffold; baseline (speedup 1.0000x reference)
import jax
import jax.numpy as jnp
from jax.experimental import pallas as pl


def kernel(node_features, edge_index, claim_node_idx, W_in, b_in, W0, b0, W1, b1, W2, b2, Wc1, bc1, Wc2, bc2):
    raise NotImplementedError("write your pallas kernel here")


# speedup vs baseline: 4.9403x; 4.9403x over previous
import functools

import jax
import jax.numpy as jnp
from jax import lax
from jax.experimental import pallas as pl
from jax.experimental.pallas import tpu as pltpu
from jax.experimental.pallas import tpu_sc as plsc

NTILES = 16
NCORES = 2
CHUNK = 128
CNTW = 128


def _sc_degree_kernel(np_rows, nchunk):
    stripe = np_rows // NTILES
    nzcopy = stripe // CHUNK

    def body(idx_hbm, cnt_hbm, idx_v, work_v, cnt_sh):
        c = lax.axis_index("c")
        s = lax.axis_index("s")
        pltpu.sync_copy(idx_hbm.at[c, s], idx_v)
        zv = jnp.zeros((16,), jnp.float32)

        def zero_body(i, _):
            work_v[i // (CNTW // 16), pl.ds((i % (CNTW // 16)) * 16, 16)] = zv
            return 0

        lax.fori_loop(0, CHUNK * (CNTW // 16), zero_body, 0)
        for k in range(nzcopy):
            pltpu.sync_copy(
                work_v, cnt_sh.at[pl.ds(s * stripe + k * CHUNK, CHUNK)])
        ov = jnp.ones((16,), jnp.float32)

        def one_body(i, _):
            work_v[i // (CNTW // 16), pl.ds((i % (CNTW // 16)) * 16, 16)] = ov
            return 0

        lax.fori_loop(0, CHUNK * (CNTW // 16), one_body, 0)
        plsc.subcore_barrier()

        def sc_body(j, _):
            pltpu.sync_copy(work_v, cnt_sh.at[idx_v.at[j]], add=True)
            return 0

        lax.fori_loop(0, nchunk, sc_body, 0)
        plsc.subcore_barrier()
        pltpu.sync_copy(cnt_sh.at[pl.ds(s * stripe, stripe)],
                        cnt_hbm.at[c, pl.ds(s * stripe, stripe)])

    return pl.kernel(
        body,
        out_type=jax.ShapeDtypeStruct((NCORES, np_rows, CNTW), jnp.float32),
        mesh=plsc.VectorSubcoreMesh(core_axis_name="c", subcore_axis_name="s"),
        scratch_types=[
            pltpu.VMEM((nchunk, CHUNK), jnp.int32),
            pltpu.VMEM((CHUNK, CNTW), jnp.float32),
            pltpu.VMEM_SHARED((np_rows, CNTW), jnp.float32),
        ],
    )


def _sc_agg_kernel(np_rows, nchunk, hh):
    stripe = np_rows // NTILES
    nzcopy = stripe // CHUNK

    def body(table_hbm, idxg_hbm, idxs_hbm, agg_hbm,
             idxg_v, idxs_v, rows_v, acc_sh, sem):
        c = lax.axis_index("c")
        s = lax.axis_index("s")
        pltpu.sync_copy(idxg_hbm.at[c, s], idxg_v)
        pltpu.sync_copy(idxs_hbm.at[s], idxs_v)
        zv = jnp.zeros((16,), jnp.float32)

        def zero_body(i, _):
            rows_v[i // (hh // 16), pl.ds((i % (hh // 16)) * 16, 16)] = zv
            return 0

        lax.fori_loop(0, CHUNK * (hh // 16), zero_body, 0)
        for k in range(nzcopy):
            pltpu.sync_copy(
                rows_v, acc_sh.at[pl.ds(s * stripe + k * CHUNK, CHUNK)])
        plsc.subcore_barrier()

        def chunk_body(j, _):
            pltpu.async_copy(table_hbm.at[idxg_v.at[j]], rows_v, sem).wait()
            pltpu.sync_copy(rows_v, acc_sh.at[idxs_v.at[j]], add=True)
            return 0

        lax.fori_loop(0, nchunk, chunk_body, 0)
        plsc.subcore_barrier()
        pltpu.sync_copy(acc_sh.at[pl.ds(s * stripe, stripe)],
                        agg_hbm.at[c, pl.ds(s * stripe, stripe)])

    return pl.kernel(
        body,
        out_type=jax.ShapeDtypeStruct((NCORES, np_rows, hh), jnp.float32),
        mesh=plsc.VectorSubcoreMesh(core_axis_name="c", subcore_axis_name="s"),
        scratch_types=[
            pltpu.VMEM((nchunk, CHUNK), jnp.int32),
            pltpu.VMEM((nchunk, CHUNK), jnp.int32),
            pltpu.VMEM((CHUNK, hh), jnp.float32),
            pltpu.VMEM_SHARED((np_rows, hh), jnp.float32),
            pltpu.SemaphoreType.DMA,
        ],
    )


def _proj_body(hh, x_ref, w_ref, b_ref, cnt_ref, out_ref):
    inv_out = lax.rsqrt(jnp.maximum(cnt_ref[0][:, 0:1], 1.0))
    h = jnp.dot(x_ref[...], w_ref[...],
                preferred_element_type=jnp.float32) + b_ref[...]
    hs = h * inv_out
    out_ref[0] = hs[:, :hh]
    out_ref[1] = hs[:, hh:]


def _layer_body(hh, a0_ref, a1_ref, cd_ref, cs_ref, w_ref, b_ref, out_ref):
    inv_in = lax.rsqrt(jnp.maximum(cd_ref[0][:, 0:1], 1.0))
    inv_out = lax.rsqrt(jnp.maximum(cs_ref[0][:, 0:1], 1.0))
    a0 = a0_ref[0] * inv_in
    a1 = a1_ref[0] * inv_in
    t = (jnp.dot(a0, w_ref[:hh, :], preferred_element_type=jnp.float32)
         + jnp.dot(a1, w_ref[hh:, :], preferred_element_type=jnp.float32)
         + b_ref[...])
    hs = jnp.maximum(t, 0.0) * inv_out
    out_ref[0] = hs[:, :hh]
    out_ref[1] = hs[:, hh:]


def _head_body(n, hh, claim_ref, agg_ref, cnt_ref,
               w2_ref, b2_ref, wc1_ref, bc1_ref, wc2_ref, bc2_ref, out_ref):
    claim = claim_ref[0, 0]
    rows = lax.broadcasted_iota(jnp.int32, (n, 1), 0)
    msk = (rows == claim).astype(jnp.float32)
    a0 = jnp.sum(agg_ref[0] * msk, axis=0, keepdims=True)
    a1 = jnp.sum(agg_ref[1] * msk, axis=0, keepdims=True)
    cnt = jnp.sum(cnt_ref[0][:, 0:1] * msk, axis=0, keepdims=True)
    inv_in = lax.rsqrt(jnp.maximum(cnt, 1.0))
    t = (jnp.dot(a0, w2_ref[:hh, :], preferred_element_type=jnp.float32)
         + jnp.dot(a1, w2_ref[hh:, :], preferred_element_type=jnp.float32))
    h2 = jnp.maximum(t * inv_in + b2_ref[...], 0.0)
    hid = jnp.maximum(
        jnp.dot(h2, wc1_ref[...], preferred_element_type=jnp.float32)
        + bc1_ref[...], 0.0)
    out_ref[...] = (jnp.dot(hid, wc2_ref[...],
                            preferred_element_type=jnp.float32) + bc2_ref[...])


def kernel(node_features, edge_index, claim_node_idx,
           W_in, b_in, W0, b0, W1, b1, W2, b2, Wc1, bc1, Wc2, bc2):
    n, d = node_features.shape
    e = edge_index.shape[1]
    h = W_in.shape[1]
    hh = h // 2
    c_out = Wc2.shape[1]
    pad_row = n
    blk = 1000 if n % 1000 == 0 else 8 * max(1, n // (8 * 10))
    np_rows = -(-(n + 8) // (NTILES * CHUNK)) * (NTILES * CHUNK)
    ept = -(-e // NTILES)
    nchunk = -(-ept // CHUNK)
    tot = NTILES * nchunk * CHUNK

    src = edge_index[0]
    dst = edge_index[1]
    pad = jnp.full((tot - e,), pad_row, jnp.int32)
    srcp = jnp.concatenate([src, pad])
    dstp = jnp.concatenate([dst, pad])
    idx_deg = jnp.stack([srcp, dstp]).reshape(NCORES, NTILES, nchunk, CHUNK)
    idx_gather = jnp.stack([srcp, srcp + np_rows]).reshape(
        NCORES, NTILES, nchunk, CHUNK)
    idx_scatter = dstp.reshape(NTILES, nchunk, CHUNK)
    claim = jnp.asarray(claim_node_idx, jnp.int32).reshape(1, 1)
    b_in2 = b_in.reshape(1, h)

    counts = _sc_degree_kernel(np_rows, nchunk)(idx_deg)

    grid = n // blk
    cnt_spec = lambda which: pl.BlockSpec(
        (1, blk, CNTW), lambda r: (which, r, 0))
    half_spec = lambda which: pl.BlockSpec(
        (1, blk, hh), lambda r: (which, r, 0))
    out_spec = pl.BlockSpec((2, blk, hh), lambda r: (0, r, 0))
    full = lambda a, b: pl.BlockSpec((a, b), lambda r: (0, 0))

    hs = pl.pallas_call(
        functools.partial(_proj_body, hh),
        grid=(grid,),
        in_specs=[
            pl.BlockSpec((blk, d), lambda r: (r, 0)),
            full(d, h),
            full(1, h),
            cnt_spec(0),
        ],
        out_specs=out_spec,
        out_shape=jax.ShapeDtypeStruct((NCORES, np_rows, hh), jnp.float32),
    )(node_features, W_in, b_in2, counts)

    agg_fn = _sc_agg_kernel(np_rows, nchunk, hh)
    layer_fn = pl.pallas_call(
        functools.partial(_layer_body, hh),
        grid=(grid,),
        in_specs=[
            half_spec(0),
            half_spec(1),
            cnt_spec(1),
            cnt_spec(0),
            full(h, h),
            full(1, h),
        ],
        out_specs=out_spec,
        out_shape=jax.ShapeDtypeStruct((NCORES, np_rows, hh), jnp.float32),
    )

    for w, b in ((W0, b0), (W1, b1)):
        agg = agg_fn(hs.reshape(NCORES * np_rows, hh),
                     idx_gather, idx_scatter)
        hs = layer_fn(agg, agg, counts, counts, w, b.reshape(1, h))

    agg = agg_fn(hs.reshape(NCORES * np_rows, hh), idx_gather, idx_scatter)

    logits = pl.pallas_call(
        functools.partial(_head_body, n, hh),
        grid=(1,),
        in_specs=[
            pl.BlockSpec(memory_space=pltpu.SMEM),
            pl.BlockSpec((2, n, hh), lambda r: (0, 0, 0)),
            pl.BlockSpec((1, n, CNTW), lambda r: (1, 0, 0)),
            full(h, h),
            full(1, h),
            full(h, hh),
            full(1, hh),
            full(hh, c_out),
            full(1, c_out),
        ],
        out_specs=pl.BlockSpec((1, c_out), lambda r: (0, 0)),
        out_shape=jax.ShapeDtypeStruct((1, c_out), jnp.float32),
    )(claim, agg, counts, W2, b2.reshape(1, h), Wc1, bc1.reshape(1, hh),
      Wc2, bc2.reshape(1, c_out))
    return logits[0]
